# baseline (speedup 1.0000x reference)
import jax
import jax.numpy as jnp
from jax import lax
from jax.experimental import pallas as pl
from jax.experimental.pallas import tpu as pltpu

_BATCHER8 = [
    (0, 1), (2, 3), (4, 5), (6, 7),
    (0, 2), (1, 3), (4, 6), (5, 7),
    (1, 2), (5, 6),
    (0, 4), (1, 5), (2, 6), (3, 7),
    (2, 4), (3, 5),
    (1, 2), (3, 4), (5, 6),
]

_BITONIC8 = [
    (0, 4), (1, 5), (2, 6), (3, 7),
    (0, 2), (1, 3), (4, 6), (5, 7),
    (0, 1), (2, 3), (4, 5), (6, 7),
]


def _ce(a, i, j):
    hi = jnp.maximum(a[i], a[j])
    lo = jnp.minimum(a[i], a[j])
    a[i] = hi
    a[j] = lo


def _merge_keep_top(a, b):
    m = [jnp.maximum(a[i], b[7 - i]) for i in range(8)]
    for (i, j) in _BITONIC8:
        _ce(m, i, j)
    return m


def _sorted_group(x_ref, b, start):
    g = [x_ref[b, pl.ds(pl.multiple_of(start + 8 * k, 8), 8), :]
         for k in range(8)]
    for (i, j) in _BATCHER8:
        _ce(g, i, j)
    return g


def _body(x_ref, o_ref):
    NB, S, C = x_ref.shape
    niter = S // 256

    for b in range(NB):
        neg = jnp.full((8, C), -jnp.inf, jnp.float32)

        def step(m, carry):
            base = m * 256
            return tuple(
                tuple(_merge_keep_top(list(st),
                                      _sorted_group(x_ref, b, base + 64 * i)))
                for i, st in enumerate(carry))

        sts = lax.fori_loop(
            0, niter, step, ((tuple([neg] * 8),) * 4), unroll=8)
        c = _merge_keep_top(
            _merge_keep_top(list(sts[0]), list(sts[1])),
            _merge_keep_top(list(sts[2]), list(sts[3])))
        for shift in (4, 2, 1):
            rolled = [pltpu.roll(c[k], shift, axis=0) for k in range(8)]
            c = _merge_keep_top(c, rolled)
        for k in range(8):
            o_ref[b, k:k + 1, :] = c[k][0:1, :]


def kernel(x):
    B, S, C = x.shape
    NB = 4
    out = pl.pallas_call(
        _body,
        grid=(B // NB,),
        in_specs=[pl.BlockSpec((NB, S, C), lambda b: (b, 0, 0))],
        out_specs=pl.BlockSpec((NB, 8, C), lambda b: (b, 0, 0)),
        out_shape=jax.ShapeDtypeStruct((B, 8, C), x.dtype),
    )(x)
    return out.reshape(B, 8 * C)


# speedup vs baseline: 1.0137x; 1.0137x over previous
import jax
import jax.numpy as jnp
from jax import lax
from jax.experimental import pallas as pl
from jax.experimental.pallas import tpu as pltpu

_BATCHER8 = [
    (0, 1), (2, 3), (4, 5), (6, 7),
    (0, 2), (1, 3), (4, 6), (5, 7),
    (1, 2), (5, 6),
    (0, 4), (1, 5), (2, 6), (3, 7),
    (2, 4), (3, 5),
    (1, 2), (3, 4), (5, 6),
]

_BITONIC8 = [
    (0, 4), (1, 5), (2, 6), (3, 7),
    (0, 2), (1, 3), (4, 6), (5, 7),
    (0, 1), (2, 3), (4, 5), (6, 7),
]


def _ce(a, i, j):
    hi = jnp.maximum(a[i], a[j])
    lo = jnp.minimum(a[i], a[j])
    a[i] = hi
    a[j] = lo


def _merge_keep_top(a, b):
    m = [jnp.maximum(a[i], b[7 - i]) for i in range(8)]
    for (i, j) in _BITONIC8:
        _ce(m, i, j)
    return m


def _sorted_group(x_ref, b, start):
    g = [x_ref[b, pl.ds(pl.multiple_of(start + 8 * k, 8), 8), :]
         for k in range(8)]
    for (i, j) in _BATCHER8:
        _ce(g, i, j)
    return g


def _body(x_ref, o_ref):
    NB, S, C = x_ref.shape
    niter = S // 128

    for b in range(NB):
        neg = jnp.full((8, C), -jnp.inf, jnp.float32)

        def step(m, carry):
            st0, st1 = carry
            base = m * 128
            st0 = _merge_keep_top(list(st0), _sorted_group(x_ref, b, base))
            st1 = _merge_keep_top(list(st1), _sorted_group(x_ref, b, base + 64))
            return (tuple(st0), tuple(st1))

        st0, st1 = lax.fori_loop(
            0, niter, step, (tuple([neg] * 8), tuple([neg] * 8)), unroll=16)
        c = _merge_keep_top(list(st0), list(st1))
        for shift in (4, 2, 1):
            rolled = [pltpu.roll(c[k], shift, axis=0) for k in range(8)]
            c = _merge_keep_top(c, rolled)
        for k in range(8):
            o_ref[b, k:k + 1, :] = c[k][0:1, :]


def kernel(x):
    B, S, C = x.shape
    NB = 4
    out = pl.pallas_call(
        _body,
        grid=(B // NB,),
        in_specs=[pl.BlockSpec((NB, S, C), lambda b: (b, 0, 0))],
        out_specs=pl.BlockSpec((NB, 8, C), lambda b: (b, 0, 0)),
        out_shape=jax.ShapeDtypeStruct((B, 8, C), x.dtype),
    )(x)
    return out.reshape(B, 8 * C)
